# baseline (speedup 1.0000x reference)
import functools

import jax
import jax.numpy as jnp
from jax import lax
from jax.experimental import pallas as pl
from jax.experimental.pallas import tpu as pltpu
from jax.experimental.pallas import tpu_sc as plsc

N = 1048576
NC = 2
NS = 16
NW = NC * NS
L = 16
RW = N // NW

RTC = 16384
CH = 4096
CHR = CH // 128
NCH = RW // CH
NR = N // 128

B1 = 512
B2 = 512
BE = 768
DUMP2 = 600

CLS_W = 1.0
BBOX_W = 0.5
LMK_W = 0.5

_G_COEF = (1.8498544538905285e-04, 2.8751506391739456e-04,
           -5.4268610571399910e-03, 8.3107776364009530e-05,
           1.2498464620813230e-01, -4.9999884358222030e-01,
           6.9314715967354310e-01)

_MESH = plsc.VectorSubcoreMesh(core_axis_name="c", subcore_axis_name="s")
_CPARAMS = pltpu.CompilerParams(needs_layout_passes=False)


def _tc_body(pred_ref, off_ref, lmk_ref, vz_ref, vo_ref, vl_ref):
    pt = pred_ref[...]
    ot = off_ref[...]
    lt = lmk_ref[...]
    do = pt[1:5, :] - ot
    dl = pt[5:15, :] - lt
    vz_ref[...] = pt[0, :]
    vo_ref[...] = jnp.sum(do * do, axis=0)
    vl_ref[...] = jnp.sum(dl * dl, axis=0)


_tc_values = pl.pallas_call(
    _tc_body,
    grid=(N // RTC,),
    in_specs=[
        pl.BlockSpec((15, RTC), lambda i: (0, i)),
        pl.BlockSpec((4, RTC), lambda i: (0, i)),
        pl.BlockSpec((10, RTC), lambda i: (0, i)),
    ],
    out_specs=[pl.BlockSpec((RTC,), lambda i: (i,))] * 3,
    out_shape=[jax.ShapeDtypeStruct((N,), jnp.float32)] * 3,
)


def _wid():
    return lax.axis_index("s") * NC + lax.axis_index("c")


def _g_poly(s):
    acc = jnp.full(s.shape, _G_COEF[0], jnp.float32)
    for c in _G_COEF[1:]:
        acc = acc * s + c
    return acc


def _zero_ref(ref, nwords):
    z = jnp.zeros((L,), jnp.float32)

    @pl.loop(0, nwords // L)
    def _(i):
        ref[pl.ds(i * L, L)] = z


def _accum_rows(src_hbm, stage, acc, nwords, nrows):
    _zero_ref(acc, nwords)

    @pl.loop(0, nrows)
    def _(t):
        pltpu.sync_copy(src_hbm.at[pl.ds(t * nwords, nwords)], stage)

        @pl.loop(0, nwords // L)
        def _(i):
            sl = pl.ds(i * L, L)
            acc[sl] = acc[sl] + stage[sl]


def _scan_top(ref, cnt_base, sum_base, nbins, target):
    nb = nbins // L

    def body(j, carry):
        found, b_star, s_above, c_above, ccnt, csum = carry
        vb = nb - 1 - j
        vc = ref[pl.ds(cnt_base + vb * L, L)]
        vs = ref[pl.ds(sum_base + vb * L, L)]
        rc = lax.rev(vc, (0,))
        rs = lax.rev(vs, (0,))
        cum = jnp.cumsum(rc) + ccnt
        m = cum >= target
        p = jnp.sum(jnp.where(m, 1.0, 0.0))
        has = (p > 0.5).astype(jnp.int32)
        b_here = vb * L + lax.convert_element_type(p, jnp.int32) - 1
        c_here = ccnt + jnp.sum(jnp.where(m, 0.0, rc))
        s_here = csum + jnp.sum(jnp.where(m, 0.0, rs))
        take = has * (1 - found)
        b_star = jnp.where(take > 0, b_here, b_star)
        s_above = jnp.where(take > 0, s_here, s_above)
        c_above = jnp.where(take > 0, c_here, c_above)
        found = jnp.maximum(found, has)
        ccnt = ccnt + jnp.sum(vc)
        csum = csum + jnp.sum(vs)
        return (found, b_star, s_above, c_above, ccnt, csum)

    init = (jnp.int32(0), jnp.int32(0), jnp.float32(0.0), jnp.float32(0.0),
            jnp.float32(0.0), jnp.float32(0.0))
    _, b_star, s_above, c_above, _, _ = lax.fori_loop(0, nb, body, init)
    return b_star, s_above, c_above


def _hist_count(ref, cnt_base, nbins):
    acc = jnp.zeros((L,), jnp.float32)

    def body(i, acc):
        return acc + ref[pl.ds(cnt_base + i * L, L)]

    acc = lax.fori_loop(0, nbins // L, body, acc)
    return jnp.sum(acc)


def _n_keep(count_f):
    ci = lax.convert_element_type(count_f, jnp.int32)
    nk = (7 * ci) // 10
    return lax.convert_element_type(nk, jnp.float32)


def _sdiv(a, b):
    q = jnp.full((L,), a, jnp.float32) / jnp.full((L,), b, jnp.float32)
    lane = lax.iota(jnp.int32, L)
    return jnp.sum(jnp.where(lane == 0, q, jnp.zeros((L,), jnp.float32)))


def _scalar_at(ref, idx):
    v = plsc.load_gather(ref, [jnp.full((L,), idx, jnp.int32)])
    return jnp.sum(v) * (1.0 / L)


def _fill_ones(ref):
    o = jnp.ones((L,), jnp.float32)

    @pl.loop(0, CH // L)
    def _(i):
        ref[pl.ds(i * L, L)] = o


def _zero_spmem_hists(sid, zb_v, hists):
    @pl.when(sid == 0)
    def _():
        _zero_ref(zb_v, BE)
        for h in hists:
            pltpu.sync_copy(zb_v, h)

    plsc.subcore_barrier()


def _write_spmem_hists(sid, cid, hists, out_hbm):
    plsc.subcore_barrier()

    @pl.when(sid == 0)
    def _():
        for j, h in enumerate(hists):
            pltpu.sync_copy(
                h, out_hbm.at[pl.ds((cid * 6 + j) * BE, BE)])


@functools.partial(
    pl.kernel,
    out_type=(
        jax.ShapeDtypeStruct((N,), jnp.float32),
        jax.ShapeDtypeStruct((N,), jnp.float32),
        jax.ShapeDtypeStruct((N,), jnp.float32),
        jax.ShapeDtypeStruct((NC * 6 * BE,), jnp.float32),
    ),
    mesh=_MESH,
    compiler_params=_CPARAMS,
    scratch_types=(
        (pltpu.VMEM((CH,), jnp.int32),) * 2,
        (pltpu.VMEM((CH,), jnp.float32),) * 2,
        (pltpu.VMEM((CH,), jnp.float32),) * 2,
        (pltpu.VMEM((CH,), jnp.float32),) * 2,
        (pltpu.VMEM((CH,), jnp.float32),) * 2,
        (pltpu.VMEM((CH,), jnp.float32),) * 2,
        (pltpu.VMEM((CH,), jnp.float32),) * 2,
        (pltpu.VMEM((CH,), jnp.int32),) * 2,
        (pltpu.VMEM((CH,), jnp.int32),) * 2,
        (pltpu.VMEM((CH,), jnp.int32),) * 2,
        pltpu.VMEM((CH,), jnp.float32),
        pltpu.VMEM((BE,), jnp.float32),
        (pltpu.VMEM_SHARED((BE,), jnp.float32),) * 6,
        (pltpu.SemaphoreType.DMA,) * 2,
        (pltpu.SemaphoreType.DMA,) * 2,
    ),
)
def _kernel_h1(lab_hbm, vz_hbm, vso_hbm, vsl_hbm,
               vc_hbm, vo_hbm, vl_hbm, h1_hbm,
               lab_b, z_b, so_b, sl_b, oc_b, oo_b, ol_b,
               ic_b, io_b, il_b, ones_v, zb_v, hsh, semi, semo):
    wid = _wid()
    sid = lax.axis_index("s")
    cid = lax.axis_index("c")
    neg1 = jnp.full((L,), -1.0, jnp.float32)
    c22 = jnp.full((L,), 22, jnp.int32)

    in_pairs = ((lab_hbm, lab_b), (vz_hbm, z_b), (vso_hbm, so_b),
                (vsl_hbm, sl_b))
    out_pairs = ((oc_b, vc_hbm), (oo_b, vo_hbm), (ol_b, vl_hbm))

    def start_in(ci, b):
        row0 = wid * RW + ci * CH
        for hbm, buf in in_pairs:
            pltpu.async_copy(hbm.at[pl.ds(row0, CH)], buf[b], semi[b])

    def wait_in(b):
        for hbm, buf in in_pairs:
            pltpu.make_async_copy(hbm.at[pl.ds(0, CH)], buf[b],
                                  semi[b]).wait()

    def start_out(ci, b):
        row0 = wid * RW + ci * CH
        for buf, hbm in out_pairs:
            pltpu.async_copy(buf[b], hbm.at[pl.ds(row0, CH)], semo[b])

    def wait_out(b):
        for buf, hbm in out_pairs:
            pltpu.make_async_copy(buf[b], hbm.at[pl.ds(0, CH)],
                                  semo[b]).wait()

    _fill_ones(ones_v)
    _zero_spmem_hists(sid, zb_v, hsh)
    start_in(0, 0)

    @pl.loop(0, NCH // 2)
    def _(oc):
        for b in range(2):
            ci = oc * 2 + b
            wait_in(b)

            @pl.when(ci + 1 < NCH)
            def _():
                start_in(ci + 1, 1 - b)

            @pl.when(ci >= 2)
            def _():
                wait_out(b)

            @pl.loop(0, CH // L, unroll=2)
            def _(g):
                sl = pl.ds(g * L, L)
                lbl = lab_b[b][sl]
                z = z_b[b][sl]
                so = so_b[b][sl]
                sl10 = sl_b[b][sl]

                s = 1.0 / (1.0 + jnp.exp(-z))
                y = jnp.where(lbl == 1, 1.0, 0.0)
                per_cls = s * (1.0 - y) + _g_poly(s)
                keep = lbl >= 0
                per_off = so * 0.25
                offm = (lbl == 1) | (lbl == -1)
                per_lmk = sl10 * 0.1
                lmkm = lbl == -2

                for (per, msk, obuf, ibuf) in (
                        (per_cls, keep, oc_b, ic_b),
                        (per_off, offm, oo_b, io_b),
                        (per_lmk, lmkm, ol_b, il_b)):
                    v = jnp.where(msk, per, neg1)
                    obuf[b][sl] = v
                    bits = plsc.bitcast(v, jnp.int32)
                    ibuf[b][sl] = lax.shift_right_logical(bits, c22)

            start_out(ci, b)
            for k, (obuf, ibuf) in enumerate(
                    ((oc_b, ic_b), (oo_b, io_b), (ol_b, il_b))):
                pltpu.sync_copy(ones_v, hsh[2 * k].at[ibuf[b]], add=True)
                pltpu.sync_copy(obuf[b], hsh[2 * k + 1].at[ibuf[b]],
                                add=True)

    for b in range(2):
        wait_out(b)
    _write_spmem_hists(sid, cid, hsh, h1_hbm)


@functools.partial(
    pl.kernel,
    out_type=jax.ShapeDtypeStruct((NC * 6 * BE,), jnp.float32),
    mesh=_MESH,
    compiler_params=_CPARAMS,
    scratch_types=(
        pltpu.VMEM((6 * BE,), jnp.float32),
        pltpu.VMEM((6 * BE,), jnp.float32),
        (pltpu.VMEM((CH,), jnp.float32),) * 2,
        (pltpu.VMEM((CH,), jnp.float32),) * 2,
        (pltpu.VMEM((CH,), jnp.float32),) * 2,
        (pltpu.VMEM((CH,), jnp.int32),) * 2,
        (pltpu.VMEM((CH,), jnp.int32),) * 2,
        (pltpu.VMEM((CH,), jnp.int32),) * 2,
        pltpu.VMEM((CH,), jnp.float32),
        pltpu.VMEM((BE,), jnp.float32),
        (pltpu.VMEM_SHARED((BE,), jnp.float32),) * 6,
        (pltpu.SemaphoreType.DMA,) * 2,
    ),
)
def _kernel_h2(vc_hbm, vo_hbm, vl_hbm, h1_hbm, h2_hbm,
               acc1_v, st1_v, bc_b, bo_b, bl_b,
               ic_b, io_b, il_b, ones_v, zb_v, hsh, semi):
    wid = _wid()
    sid = lax.axis_index("s")
    cid = lax.axis_index("c")
    c22 = jnp.full((L,), 22, jnp.int32)
    c13 = jnp.full((L,), 13, jnp.int32)
    dump = jnp.full((L,), DUMP2, jnp.int32)

    in_pairs = ((vc_hbm, bc_b), (vo_hbm, bo_b), (vl_hbm, bl_b))

    def start_in(ci, b):
        row0 = wid * RW + ci * CH
        for hbm, buf in in_pairs:
            pltpu.async_copy(hbm.at[pl.ds(row0, CH)], buf[b], semi[b])

    def wait_in(b):
        for hbm, buf in in_pairs:
            pltpu.make_async_copy(hbm.at[pl.ds(0, CH)], buf[b],
                                  semi[b]).wait()

    _accum_rows(h1_hbm, st1_v, acc1_v, 6 * BE, NC)

    b1s = []
    for k in range(3):
        count = _hist_count(acc1_v, k * 2 * BE, B1)
        nk = _n_keep(count)
        b1, _, _ = _scan_top(acc1_v, k * 2 * BE, (k * 2 + 1) * BE, B1, nk)
        b1s.append(jnp.full((L,), b1, jnp.int32))

    _fill_ones(ones_v)
    _zero_spmem_hists(sid, zb_v, hsh)
    start_in(0, 0)

    @pl.loop(0, NCH // 2)
    def _(oc):
        for b in range(2):
            ci = oc * 2 + b
            wait_in(b)

            @pl.when(ci + 1 < NCH)
            def _():
                start_in(ci + 1, 1 - b)

            @pl.loop(0, CH // L, unroll=2)
            def _(g):
                sl = pl.ds(g * L, L)
                for k, (vbuf, ibuf) in enumerate(
                        ((bc_b, ic_b), (bo_b, io_b), (bl_b, il_b))):
                    v = vbuf[b][sl]
                    bits = plsc.bitcast(v, jnp.int32)
                    lvl1 = lax.shift_right_logical(bits, c22)
                    sub = jnp.bitwise_and(
                        lax.shift_right_logical(bits, c13), B2 - 1)
                    ibuf[b][sl] = jnp.where(lvl1 == b1s[k], sub, dump)

            for k, (vbuf, ibuf) in enumerate(
                    ((bc_b, ic_b), (bo_b, io_b), (bl_b, il_b))):
                pltpu.sync_copy(ones_v, hsh[2 * k].at[ibuf[b]], add=True)
                pltpu.sync_copy(vbuf[b], hsh[2 * k + 1].at[ibuf[b]],
                                add=True)

    _write_spmem_hists(sid, cid, hsh, h2_hbm)


@functools.partial(
    pl.kernel,
    out_type=jax.ShapeDtypeStruct((8,), jnp.float32),
    mesh=_MESH,
    compiler_params=_CPARAMS,
    scratch_types=(
        pltpu.VMEM((6 * BE,), jnp.float32),
        pltpu.VMEM((6 * BE,), jnp.float32),
        pltpu.VMEM((6 * BE,), jnp.float32),
        pltpu.VMEM((16,), jnp.float32),
    ),
)
def _kernel_c(h1_hbm, h2_hbm, out_hbm, acc1_v, acc2_v, st_v, out_v):
    wid = _wid()

    @pl.when(wid == 0)
    def _():
        _accum_rows(h1_hbm, st_v, acc1_v, 6 * BE, NC)
        _accum_rows(h2_hbm, st_v, acc2_v, 6 * BE, NC)

        losses = []
        for k in range(3):
            count = _hist_count(acc1_v, k * 2 * BE, B1)
            nk = _n_keep(count)
            _, s1, c1 = _scan_top(acc1_v, k * 2 * BE, (k * 2 + 1) * BE,
                                  B1, nk)
            r1 = nk - c1
            b2, s2, c2 = _scan_top(acc2_v, k * 2 * BE, (k * 2 + 1) * BE,
                                   B2, r1)
            r2 = r1 - c2
            cnt_b2 = _scalar_at(acc2_v, k * 2 * BE + b2)
            sum_b2 = _scalar_at(acc2_v, (k * 2 + 1) * BE + b2)
            part = jnp.where(r2 > 0.5, r2 * _sdiv(sum_b2, cnt_b2), 0.0)
            total = s1 + s2 + part
            mean = _sdiv(total, nk)
            if k == 0:
                losses.append(mean)
            else:
                losses.append(jnp.where(count < 0.5, 0.0, mean))

        loss_cls, loss_off, loss_lmk = losses
        loss_total = CLS_W * loss_cls + BBOX_W * loss_off + LMK_W * loss_lmk
        lane = lax.iota(jnp.int32, L)
        zeros = jnp.zeros((L,), jnp.float32)
        ov = jnp.where(lane == 0, loss_total, zeros)
        ov = ov + jnp.where(lane == 1, loss_cls, zeros)
        ov = ov + jnp.where(lane == 2, loss_off, zeros)
        ov = ov + jnp.where(lane == 3, loss_lmk, zeros)
        out_v[pl.ds(0, L)] = ov
        pltpu.sync_copy(out_v.at[pl.ds(0, 8)], out_hbm)


def kernel(pred, labels, offsets, landmarks):
    vz, vso, vsl = _tc_values(pred.T, offsets.T, landmarks.T)
    vc, vo, vl, h1 = _kernel_h1(labels, vz, vso, vsl)
    h2 = _kernel_h2(vc, vo, vl, h1)
    out = _kernel_c(h1, h2)
    return (out[0], out[1], out[2], out[3])


# speedup vs baseline: 13.4931x; 13.4931x over previous
import functools

import jax
import jax.numpy as jnp
import numpy as np
from jax import lax
from jax.experimental import pallas as pl
from jax.experimental.pallas import tpu as pltpu
from jax.experimental.pallas import tpu_sc as plsc

N = 1048576
NC = 2
NS = 16
NW = NC * NS
L = 16
RW = N // NW

RTC = 16384
CH = 4096
NCH = RW // CH

B1 = 1024
B2 = 512
HR1 = 3 * B1
HR2 = 3 * 2 * B2 + 16

CLS_W = 1.0
BBOX_W = 0.5
LMK_W = 0.5

_G_COEF = (1.8498544538905285e-04, 2.8751506391739456e-04,
           -5.4268610571399910e-03, 8.3107776364009530e-05,
           1.2498464620813230e-01, -4.9999884358222030e-01,
           6.9314715967354310e-01)

_MESH = plsc.VectorSubcoreMesh(core_axis_name="c", subcore_axis_name="s")
_CPARAMS = pltpu.CompilerParams(needs_layout_passes=False)


def _tc_body(pred_ref, off_ref, lmk_ref, vz_ref, vo_ref, vl_ref):
    pt = pred_ref[...]
    ot = off_ref[...]
    lt = lmk_ref[...]
    do = pt[1:5, :] - ot
    dl = pt[5:15, :] - lt
    vz_ref[...] = pt[0, :]
    vo_ref[...] = jnp.sum(do * do, axis=0)
    vl_ref[...] = jnp.sum(dl * dl, axis=0)


_tc_values = pl.pallas_call(
    _tc_body,
    grid=(N // RTC,),
    in_specs=[
        pl.BlockSpec((15, RTC), lambda i: (0, i)),
        pl.BlockSpec((4, RTC), lambda i: (0, i)),
        pl.BlockSpec((10, RTC), lambda i: (0, i)),
    ],
    out_specs=[pl.BlockSpec((RTC,), lambda i: (i,))] * 3,
    out_shape=[jax.ShapeDtypeStruct((N,), jnp.float32)] * 3,
)


def _wid():
    return lax.axis_index("s") * NC + lax.axis_index("c")


def _g_poly(s):
    acc = jnp.full(s.shape, _G_COEF[0], jnp.float32)
    for c in _G_COEF[1:]:
        acc = acc * s + c
    return acc


def _zero_ref(ref, nwords):
    z = jnp.zeros((L,), jnp.float32)

    @pl.loop(0, nwords // L)
    def _(i):
        ref[pl.ds(i * L, L)] = z


def _lane_fold(src, src_base, dst, dst_base, nbins, lane):
    @pl.loop(0, nbins // L)
    def _(i):
        bins = i * L + lane
        acc = jnp.zeros((L,), jnp.float32)
        for l in range(L):
            acc = acc + plsc.load_gather(src, [src_base + bins * L + l])
        dst[pl.ds(dst_base + i * L, L)] = acc


def _accum_rows(src_hbm, stage, acc, nwords, nrows):
    _zero_ref(acc, nwords)

    @pl.loop(0, nrows)
    def _(t):
        pltpu.sync_copy(src_hbm.at[pl.ds(t * nwords, nwords)], stage)

        @pl.loop(0, nwords // L)
        def _(i):
            sl = pl.ds(i * L, L)
            acc[sl] = acc[sl] + stage[sl]


def _combine_per_sc(fold_v, shared, stage, acc, out_hbm, nwords):
    sid = lax.axis_index("s")
    cid = lax.axis_index("c")
    pltpu.sync_copy(fold_v, shared.at[sid])
    plsc.subcore_barrier()

    @pl.when(sid == 0)
    def _():
        _zero_ref(acc, nwords)

        @pl.loop(0, NS)
        def _(t):
            pltpu.sync_copy(shared.at[t], stage)

            @pl.loop(0, nwords // L)
            def _(i):
                sl = pl.ds(i * L, L)
                acc[sl] = acc[sl] + stage[sl]

        pltpu.sync_copy(acc.at[pl.ds(0, nwords)],
                        out_hbm.at[pl.ds(cid * nwords, nwords)])


def _scan_top(ref, cnt_base, sum_base, nbins, target):
    nb = nbins // L

    def body(j, carry):
        found, b_star, s_above, c_above, ccnt, csum = carry
        vb = nb - 1 - j
        vc = ref[pl.ds(cnt_base + vb * L, L)]
        vs = ref[pl.ds(sum_base + vb * L, L)]
        rc = lax.rev(vc, (0,))
        rs = lax.rev(vs, (0,))
        cum = jnp.cumsum(rc) + ccnt
        m = cum >= target
        p = jnp.sum(jnp.where(m, 1.0, 0.0))
        has = (p > 0.5).astype(jnp.int32)
        b_here = vb * L + lax.convert_element_type(p, jnp.int32) - 1
        c_here = ccnt + jnp.sum(jnp.where(m, 0.0, rc))
        s_here = csum + jnp.sum(jnp.where(m, 0.0, rs))
        take = has * (1 - found)
        b_star = jnp.where(take > 0, b_here, b_star)
        s_above = jnp.where(take > 0, s_here, s_above)
        c_above = jnp.where(take > 0, c_here, c_above)
        found = jnp.maximum(found, has)
        ccnt = ccnt + jnp.sum(vc)
        csum = csum + jnp.sum(vs)
        return (found, b_star, s_above, c_above, ccnt, csum)

    init = (jnp.int32(0), jnp.int32(0), jnp.float32(0.0), jnp.float32(0.0),
            jnp.float32(0.0), jnp.float32(0.0))
    _, b_star, s_above, c_above, _, _ = lax.fori_loop(0, nb, body, init)
    return b_star, s_above, c_above


def _scan_cnt(ref, base, nbins, target):
    nb = nbins // L

    def body(j, carry):
        found, b_star, c_above, ccnt = carry
        vb = nb - 1 - j
        vc = ref[pl.ds(base + vb * L, L)]
        rc = lax.rev(vc, (0,))
        cum = jnp.cumsum(rc) + ccnt
        m = cum >= target
        p = jnp.sum(jnp.where(m, 1.0, 0.0))
        has = (p > 0.5).astype(jnp.int32)
        b_here = vb * L + lax.convert_element_type(p, jnp.int32) - 1
        c_here = ccnt + jnp.sum(jnp.where(m, 0.0, rc))
        take = has * (1 - found)
        b_star = jnp.where(take > 0, b_here, b_star)
        c_above = jnp.where(take > 0, c_here, c_above)
        found = jnp.maximum(found, has)
        ccnt = ccnt + jnp.sum(vc)
        return (found, b_star, c_above, ccnt)

    init = (jnp.int32(0), jnp.int32(0), jnp.float32(0.0), jnp.float32(0.0))
    _, b_star, c_above, _ = lax.fori_loop(0, nb, body, init)
    return b_star, c_above


def _hist_count(ref, cnt_base, nbins):
    acc = jnp.zeros((L,), jnp.float32)

    def body(i, acc):
        return acc + ref[pl.ds(cnt_base + i * L, L)]

    acc = lax.fori_loop(0, nbins // L, body, acc)
    return jnp.sum(acc)


def _n_keep(count_f):
    ci = lax.convert_element_type(count_f, jnp.int32)
    nk = (7 * ci) // 10
    return lax.convert_element_type(nk, jnp.float32)


def _sdiv(a, b):
    q = jnp.full((L,), a, jnp.float32) / jnp.full((L,), b, jnp.float32)
    lane = lax.iota(jnp.int32, L)
    return jnp.sum(jnp.where(lane == 0, q, jnp.zeros((L,), jnp.float32)))


def _scalar_at(ref, idx):
    v = plsc.load_gather(ref, [jnp.full((L,), idx, jnp.int32)])
    return jnp.sum(v) * (1.0 / L)


@functools.partial(
    pl.kernel,
    out_type=(
        jax.ShapeDtypeStruct((N,), jnp.float32),
        jax.ShapeDtypeStruct((N,), jnp.float32),
        jax.ShapeDtypeStruct((N,), jnp.float32),
        jax.ShapeDtypeStruct((NC * HR1,), jnp.float32),
    ),
    mesh=_MESH,
    compiler_params=_CPARAMS,
    scratch_types=(
        (pltpu.VMEM((CH,), jnp.int32),) * 2,
        (pltpu.VMEM((CH,), jnp.float32),) * 2,
        (pltpu.VMEM((CH,), jnp.float32),) * 2,
        (pltpu.VMEM((CH,), jnp.float32),) * 2,
        (pltpu.VMEM((CH,), jnp.float32),) * 2,
        (pltpu.VMEM((CH,), jnp.float32),) * 2,
        (pltpu.VMEM((CH,), jnp.float32),) * 2,
        pltpu.VMEM((3 * B1 * L,), jnp.float32),
        pltpu.VMEM((HR1,), jnp.float32),
        pltpu.VMEM((HR1,), jnp.float32),
        pltpu.VMEM((HR1,), jnp.float32),
        pltpu.VMEM_SHARED((NS, HR1), jnp.float32),
        (pltpu.SemaphoreType.DMA,) * 2,
        (pltpu.SemaphoreType.DMA,) * 2,
    ),
)
def _kernel_h1(lab_hbm, vz_hbm, vso_hbm, vsl_hbm,
               vc_hbm, vo_hbm, vl_hbm, h1_hbm,
               lab_b, z_b, so_b, sl_b, oc_b, oo_b, ol_b,
               h_v, fold_v, stage_v, acc_v, shared, semi, semo):
    wid = _wid()
    lane = lax.iota(jnp.int32, L)
    ones = jnp.ones((L,), jnp.float32)
    neg1 = jnp.full((L,), -1.0, jnp.float32)
    c21 = jnp.full((L,), 21, jnp.int32)

    in_pairs = ((lab_hbm, lab_b), (vz_hbm, z_b), (vso_hbm, so_b),
                (vsl_hbm, sl_b))

    def start_in(ci, b):
        row0 = wid * RW + ci * CH
        for hbm, buf in in_pairs:
            pltpu.async_copy(hbm.at[pl.ds(row0, CH)], buf[b], semi[b])

    def wait_in(b):
        for hbm, buf in in_pairs:
            pltpu.make_async_copy(hbm.at[pl.ds(0, CH)], buf[b],
                                  semi[b]).wait()

    def start_out(ci, b):
        row0 = wid * RW + ci * CH
        pltpu.async_copy(oc_b[b], vc_hbm.at[pl.ds(row0, CH)], semo[b])
        pltpu.async_copy(oo_b[b], vo_hbm.at[pl.ds(row0, CH)], semo[b])
        pltpu.async_copy(ol_b[b], vl_hbm.at[pl.ds(row0, CH)], semo[b])

    def wait_out(b):
        for buf, hbm in ((oc_b, vc_hbm), (oo_b, vo_hbm), (ol_b, vl_hbm)):
            pltpu.make_async_copy(buf[b], hbm.at[pl.ds(0, CH)],
                                  semo[b]).wait()

    _zero_ref(h_v, 3 * B1 * L)
    start_in(0, 0)

    @pl.loop(0, NCH // 2)
    def _(oc):
        for b in range(2):
            ci = oc * 2 + b
            wait_in(b)

            @pl.when(ci + 1 < NCH)
            def _():
                start_in(ci + 1, 1 - b)

            @pl.when(ci >= 2)
            def _():
                wait_out(b)

            @pl.loop(0, CH // L, unroll=4)
            def _(g):
                sl = pl.ds(g * L, L)
                lbl = lab_b[b][sl]
                z = z_b[b][sl]
                so = so_b[b][sl]
                sl10 = sl_b[b][sl]

                s = 1.0 / (1.0 + jnp.exp(-z))
                y = jnp.where(lbl == 1, 1.0, 0.0)
                per_cls = s * (1.0 - y) + _g_poly(s)
                keep = lbl >= 0
                per_off = so * 0.25
                offm = (lbl == 1) | (lbl == -1)
                per_lmk = sl10 * 0.1
                lmkm = lbl == -2

                oc_b[b][sl] = jnp.where(keep, per_cls, neg1)
                oo_b[b][sl] = jnp.where(offm, per_off, neg1)
                ol_b[b][sl] = jnp.where(lmkm, per_lmk, neg1)

                for k, (per, msk) in enumerate(
                        ((per_cls, keep), (per_off, offm),
                         (per_lmk, lmkm))):
                    bits = plsc.bitcast(per, jnp.int32)
                    bb = lax.shift_right_logical(bits, c21)
                    idx = (k * B1 + bb) * L + lane
                    plsc.addupdate_scatter(h_v, [idx], ones, mask=msk)

            start_out(ci, b)

    for b in range(2):
        wait_out(b)

    for k in range(3):
        _lane_fold(h_v, k * B1 * L, fold_v, k * B1, B1, lane)
    _combine_per_sc(fold_v, shared, stage_v, acc_v, h1_hbm, HR1)


@functools.partial(
    pl.kernel,
    out_type=jax.ShapeDtypeStruct((NC * HR2,), jnp.float32),
    mesh=_MESH,
    compiler_params=_CPARAMS,
    scratch_types=(
        pltpu.VMEM((HR1,), jnp.float32),
        pltpu.VMEM((HR1,), jnp.float32),
        (pltpu.VMEM((CH,), jnp.float32),) * 2,
        (pltpu.VMEM((CH,), jnp.float32),) * 2,
        (pltpu.VMEM((CH,), jnp.float32),) * 2,
        pltpu.VMEM((6 * B2 * L,), jnp.float32),
        pltpu.VMEM((3 * L,), jnp.float32),
        pltpu.VMEM((HR2,), jnp.float32),
        pltpu.VMEM((HR2,), jnp.float32),
        pltpu.VMEM((HR2,), jnp.float32),
        pltpu.VMEM_SHARED((NS, HR2), jnp.float32),
        (pltpu.SemaphoreType.DMA,) * 2,
    ),
)
def _kernel_h2(vc_hbm, vo_hbm, vl_hbm, h1_hbm, h2_hbm,
               acc1_v, st1_v, bc_b, bo_b, bl_b, h2_v, sacc_v, fold_v,
               stage_v, acc_v, shared, semi):
    wid = _wid()
    lane = lax.iota(jnp.int32, L)
    ones = jnp.ones((L,), jnp.float32)
    zerov = jnp.zeros((L,), jnp.float32)
    c21 = jnp.full((L,), 21, jnp.int32)
    c12 = jnp.full((L,), 12, jnp.int32)

    def start_in(ci, b):
        row0 = wid * RW + ci * CH
        for buf, hbm in ((bc_b, vc_hbm), (bo_b, vo_hbm), (bl_b, vl_hbm)):
            pltpu.async_copy(hbm.at[pl.ds(row0, CH)], buf[b], semi[b])

    def wait_in(b):
        for buf, hbm in ((bc_b, vc_hbm), (bo_b, vo_hbm), (bl_b, vl_hbm)):
            pltpu.make_async_copy(hbm.at[pl.ds(0, CH)], buf[b],
                                  semi[b]).wait()

    _accum_rows(h1_hbm, st1_v, acc1_v, HR1, NC)

    b1s = []
    for k in range(3):
        count = _hist_count(acc1_v, k * B1, B1)
        nk = _n_keep(count)
        b1, _ = _scan_cnt(acc1_v, k * B1, B1, nk)
        b1s.append(jnp.full((L,), b1, jnp.int32))

    _zero_ref(h2_v, 6 * B2 * L)
    _zero_ref(sacc_v, 3 * L)
    start_in(0, 0)

    @pl.loop(0, NCH // 2)
    def _(oc):
        for b in range(2):
            ci = oc * 2 + b
            wait_in(b)

            @pl.when(ci + 1 < NCH)
            def _():
                start_in(ci + 1, 1 - b)

            @pl.loop(0, CH // L, unroll=2)
            def _(g):
                sl = pl.ds(g * L, L)
                for k, bufs in enumerate((bc_b, bo_b, bl_b)):
                    v = bufs[b][sl]
                    valid = v >= 0.0
                    bits = plsc.bitcast(v, jnp.int32)
                    lvl1 = lax.shift_right_logical(bits, c21)
                    m_gt = (lvl1 > b1s[k]) & valid
                    ks = pl.ds(k * L, L)
                    sacc_v[ks] = sacc_v[ks] + jnp.where(m_gt, v, zerov)
                    m_eq = lvl1 == b1s[k]
                    sub = jnp.bitwise_and(
                        lax.shift_right_logical(bits, c12), B2 - 1)
                    idx = (k * 2 * B2 + sub) * L + lane
                    plsc.addupdate_scatter(h2_v, [idx], ones, mask=m_eq)
                    plsc.addupdate_scatter(h2_v, [idx + B2 * L], v,
                                           mask=m_eq)

    for k in range(6):
        _lane_fold(h2_v, k * B2 * L, fold_v, k * B2, B2, lane)
    sv = jnp.where(lane == 0, jnp.sum(sacc_v[pl.ds(0, L)]), zerov)
    sv = sv + jnp.where(lane == 1, jnp.sum(sacc_v[pl.ds(L, L)]), zerov)
    sv = sv + jnp.where(lane == 2, jnp.sum(sacc_v[pl.ds(2 * L, L)]),
                        zerov)
    fold_v[pl.ds(6 * B2, L)] = sv
    _combine_per_sc(fold_v, shared, stage_v, acc_v, h2_hbm, HR2)


@functools.partial(
    pl.kernel,
    out_type=jax.ShapeDtypeStruct((8,), jnp.float32),
    mesh=_MESH,
    compiler_params=_CPARAMS,
    scratch_types=(
        pltpu.VMEM((HR1,), jnp.float32),
        pltpu.VMEM((HR2,), jnp.float32),
        pltpu.VMEM((HR1,), jnp.float32),
        pltpu.VMEM((HR2,), jnp.float32),
        pltpu.VMEM((16,), jnp.float32),
    ),
)
def _kernel_c(h1_hbm, h2_hbm, out_hbm, acc1_v, acc2_v, st1_v, st2_v, out_v):
    wid = _wid()

    @pl.when(wid == 0)
    def _():
        _accum_rows(h1_hbm, st1_v, acc1_v, HR1, NC)
        _accum_rows(h2_hbm, st2_v, acc2_v, HR2, NC)

        losses = []
        for k in range(3):
            count = _hist_count(acc1_v, k * B1, B1)
            nk = _n_keep(count)
            b1, c1 = _scan_cnt(acc1_v, k * B1, B1, nk)
            s1 = _scalar_at(acc2_v, 6 * B2 + k)
            r1 = nk - c1
            b2, s2, c2 = _scan_top(acc2_v, k * 2 * B2, (k * 2 + 1) * B2,
                                   B2, r1)
            r2 = r1 - c2
            cnt_b2 = _scalar_at(acc2_v, k * 2 * B2 + b2)
            sum_b2 = _scalar_at(acc2_v, (k * 2 + 1) * B2 + b2)
            part = jnp.where(r2 > 0.5, r2 * _sdiv(sum_b2, cnt_b2), 0.0)
            total = s1 + s2 + part
            mean = _sdiv(total, nk)
            if k == 0:
                losses.append(mean)
            else:
                losses.append(jnp.where(count < 0.5, 0.0, mean))

        loss_cls, loss_off, loss_lmk = losses
        loss_total = CLS_W * loss_cls + BBOX_W * loss_off + LMK_W * loss_lmk
        lane = lax.iota(jnp.int32, L)
        zeros = jnp.zeros((L,), jnp.float32)
        ov = jnp.where(lane == 0, loss_total, zeros)
        ov = ov + jnp.where(lane == 1, loss_cls, zeros)
        ov = ov + jnp.where(lane == 2, loss_off, zeros)
        ov = ov + jnp.where(lane == 3, loss_lmk, zeros)
        out_v[pl.ds(0, L)] = ov
        pltpu.sync_copy(out_v.at[pl.ds(0, 8)], out_hbm)


def kernel(pred, labels, offsets, landmarks):
    vz, vso, vsl = _tc_values(pred.T, offsets.T, landmarks.T)
    vc, vo, vl, h1 = _kernel_h1(labels, vz, vso, vsl)
    h2 = _kernel_h2(vc, vo, vl, h1)
    out = _kernel_c(h1, h2)
    return (out[0], out[1], out[2], out[3])


# speedup vs baseline: 15.4087x; 1.1420x over previous
import functools

import jax
import jax.numpy as jnp
import numpy as np
from jax import lax
from jax.experimental import pallas as pl
from jax.experimental.pallas import tpu as pltpu
from jax.experimental.pallas import tpu_sc as plsc

N = 1048576
NC = 2
NS = 16
NW = NC * NS
L = 16
RW = N // NW

RTC = 16384
CH = 4096
NCH = RW // CH

B1 = 1024
B2 = 512
HR1 = 3 * B1
HR2 = 3 * 2 * B2 + 16

CLS_W = 1.0
BBOX_W = 0.5
LMK_W = 0.5

_G_COEF = (1.8498544538905285e-04, 2.8751506391739456e-04,
           -5.4268610571399910e-03, 8.3107776364009530e-05,
           1.2498464620813230e-01, -4.9999884358222030e-01,
           6.9314715967354310e-01)

_MESH = plsc.VectorSubcoreMesh(core_axis_name="c", subcore_axis_name="s")
_CPARAMS = pltpu.CompilerParams(needs_layout_passes=False)


def _tc_body(pred_ref, off_ref, lmk_ref, va_ref, vs_ref, vo_ref, vl_ref):
    pt = pred_ref[...]
    ot = off_ref[...]
    lt = lmk_ref[...]
    do = pt[1:5, :] - ot
    dl = pt[5:15, :] - lt
    z = pt[0, :]
    sg = 1.0 / (1.0 + jnp.exp(-z))
    va_ref[...] = sg + _g_poly(sg)
    vs_ref[...] = sg
    vo_ref[...] = jnp.sum(do * do, axis=0) * 0.25
    vl_ref[...] = jnp.sum(dl * dl, axis=0) * 0.1


_tc_values = pl.pallas_call(
    _tc_body,
    grid=(N // RTC,),
    in_specs=[
        pl.BlockSpec((15, RTC), lambda i: (0, i)),
        pl.BlockSpec((4, RTC), lambda i: (0, i)),
        pl.BlockSpec((10, RTC), lambda i: (0, i)),
    ],
    out_specs=[pl.BlockSpec((RTC,), lambda i: (i,))] * 4,
    out_shape=[jax.ShapeDtypeStruct((N,), jnp.float32)] * 4,
)


def _wid():
    return lax.axis_index("s") * NC + lax.axis_index("c")


def _g_poly(s):
    acc = jnp.full(s.shape, _G_COEF[0], jnp.float32)
    for c in _G_COEF[1:]:
        acc = acc * s + c
    return acc


def _zero_ref(ref, nwords):
    z = jnp.zeros((L,), jnp.float32)

    @pl.loop(0, nwords // L)
    def _(i):
        ref[pl.ds(i * L, L)] = z


def _lane_fold(src, src_base, dst, dst_base, nbins, lane):
    @pl.loop(0, nbins // L)
    def _(i):
        bins = i * L + lane
        acc = jnp.zeros((L,), jnp.float32)
        for l in range(L):
            acc = acc + plsc.load_gather(src, [src_base + bins * L + l])
        dst[pl.ds(dst_base + i * L, L)] = acc


def _accum_rows(src_hbm, stage, acc, nwords, nrows):
    _zero_ref(acc, nwords)

    @pl.loop(0, nrows)
    def _(t):
        pltpu.sync_copy(src_hbm.at[pl.ds(t * nwords, nwords)], stage)

        @pl.loop(0, nwords // L)
        def _(i):
            sl = pl.ds(i * L, L)
            acc[sl] = acc[sl] + stage[sl]


def _combine_per_sc(fold_v, shared, stage, acc, out_hbm, nwords):
    sid = lax.axis_index("s")
    cid = lax.axis_index("c")
    pltpu.sync_copy(fold_v, shared.at[sid])
    plsc.subcore_barrier()

    @pl.when(sid == 0)
    def _():
        _zero_ref(acc, nwords)

        @pl.loop(0, NS)
        def _(t):
            pltpu.sync_copy(shared.at[t], stage)

            @pl.loop(0, nwords // L)
            def _(i):
                sl = pl.ds(i * L, L)
                acc[sl] = acc[sl] + stage[sl]

        pltpu.sync_copy(acc.at[pl.ds(0, nwords)],
                        out_hbm.at[pl.ds(cid * nwords, nwords)])


def _scan_top(ref, cnt_base, sum_base, nbins, target):
    nb = nbins // L

    def body(j, carry):
        found, b_star, s_above, c_above, ccnt, csum = carry
        vb = nb - 1 - j
        vc = ref[pl.ds(cnt_base + vb * L, L)]
        vs = ref[pl.ds(sum_base + vb * L, L)]
        rc = lax.rev(vc, (0,))
        rs = lax.rev(vs, (0,))
        cum = jnp.cumsum(rc) + ccnt
        m = cum >= target
        p = jnp.sum(jnp.where(m, 1.0, 0.0))
        has = (p > 0.5).astype(jnp.int32)
        b_here = vb * L + lax.convert_element_type(p, jnp.int32) - 1
        c_here = ccnt + jnp.sum(jnp.where(m, 0.0, rc))
        s_here = csum + jnp.sum(jnp.where(m, 0.0, rs))
        take = has * (1 - found)
        b_star = jnp.where(take > 0, b_here, b_star)
        s_above = jnp.where(take > 0, s_here, s_above)
        c_above = jnp.where(take > 0, c_here, c_above)
        found = jnp.maximum(found, has)
        ccnt = ccnt + jnp.sum(vc)
        csum = csum + jnp.sum(vs)
        return (found, b_star, s_above, c_above, ccnt, csum)

    init = (jnp.int32(0), jnp.int32(0), jnp.float32(0.0), jnp.float32(0.0),
            jnp.float32(0.0), jnp.float32(0.0))
    _, b_star, s_above, c_above, _, _ = lax.fori_loop(0, nb, body, init)
    return b_star, s_above, c_above


def _scan_cnt(ref, base, nbins, target):
    nb = nbins // L

    def body(j, carry):
        found, b_star, c_above, ccnt = carry
        vb = nb - 1 - j
        vc = ref[pl.ds(base + vb * L, L)]
        rc = lax.rev(vc, (0,))
        cum = jnp.cumsum(rc) + ccnt
        m = cum >= target
        p = jnp.sum(jnp.where(m, 1.0, 0.0))
        has = (p > 0.5).astype(jnp.int32)
        b_here = vb * L + lax.convert_element_type(p, jnp.int32) - 1
        c_here = ccnt + jnp.sum(jnp.where(m, 0.0, rc))
        take = has * (1 - found)
        b_star = jnp.where(take > 0, b_here, b_star)
        c_above = jnp.where(take > 0, c_here, c_above)
        found = jnp.maximum(found, has)
        ccnt = ccnt + jnp.sum(vc)
        return (found, b_star, c_above, ccnt)

    init = (jnp.int32(0), jnp.int32(0), jnp.float32(0.0), jnp.float32(0.0))
    _, b_star, c_above, _ = lax.fori_loop(0, nb, body, init)
    return b_star, c_above


def _hist_count(ref, cnt_base, nbins):
    acc = jnp.zeros((L,), jnp.float32)

    def body(i, acc):
        return acc + ref[pl.ds(cnt_base + i * L, L)]

    acc = lax.fori_loop(0, nbins // L, body, acc)
    return jnp.sum(acc)


def _n_keep(count_f):
    ci = lax.convert_element_type(count_f, jnp.int32)
    nk = (7 * ci) // 10
    return lax.convert_element_type(nk, jnp.float32)


def _sdiv(a, b):
    q = jnp.full((L,), a, jnp.float32) / jnp.full((L,), b, jnp.float32)
    lane = lax.iota(jnp.int32, L)
    return jnp.sum(jnp.where(lane == 0, q, jnp.zeros((L,), jnp.float32)))


def _scalar_at(ref, idx):
    v = plsc.load_gather(ref, [jnp.full((L,), idx, jnp.int32)])
    return jnp.sum(v) * (1.0 / L)


@functools.partial(
    pl.kernel,
    out_type=(
        jax.ShapeDtypeStruct((N,), jnp.float32),
        jax.ShapeDtypeStruct((N,), jnp.float32),
        jax.ShapeDtypeStruct((N,), jnp.float32),
        jax.ShapeDtypeStruct((NC * HR1,), jnp.float32),
    ),
    mesh=_MESH,
    compiler_params=_CPARAMS,
    scratch_types=(
        (pltpu.VMEM((CH,), jnp.int32),) * 2,
        (pltpu.VMEM((CH,), jnp.float32),) * 2,
        (pltpu.VMEM((CH,), jnp.float32),) * 2,
        (pltpu.VMEM((CH,), jnp.float32),) * 2,
        (pltpu.VMEM((CH,), jnp.float32),) * 2,
        (pltpu.VMEM((CH,), jnp.float32),) * 2,
        (pltpu.VMEM((CH,), jnp.float32),) * 2,
        (pltpu.VMEM((CH,), jnp.float32),) * 2,
        pltpu.VMEM((3 * B1 * L,), jnp.float32),
        pltpu.VMEM((HR1,), jnp.float32),
        pltpu.VMEM((HR1,), jnp.float32),
        pltpu.VMEM((HR1,), jnp.float32),
        pltpu.VMEM_SHARED((NS, HR1), jnp.float32),
        (pltpu.SemaphoreType.DMA,) * 2,
        (pltpu.SemaphoreType.DMA,) * 2,
    ),
)
def _kernel_h1(lab_hbm, va_hbm, vs_hbm, vso_hbm, vsl_hbm,
               vc_hbm, vo_hbm, vl_hbm, h1_hbm,
               lab_b, a_b, s_b, so_b, sl_b, oc_b, oo_b, ol_b,
               h_v, fold_v, stage_v, acc_v, shared, semi, semo):
    wid = _wid()
    lane = lax.iota(jnp.int32, L)
    ones = jnp.ones((L,), jnp.float32)
    neg1 = jnp.full((L,), -1.0, jnp.float32)
    c21 = jnp.full((L,), 21, jnp.int32)

    in_pairs = ((lab_hbm, lab_b), (va_hbm, a_b), (vs_hbm, s_b),
                (vso_hbm, so_b), (vsl_hbm, sl_b))

    def start_in(ci, b):
        row0 = wid * RW + ci * CH
        for hbm, buf in in_pairs:
            pltpu.async_copy(hbm.at[pl.ds(row0, CH)], buf[b], semi[b])

    def wait_in(b):
        for hbm, buf in in_pairs:
            pltpu.make_async_copy(hbm.at[pl.ds(0, CH)], buf[b],
                                  semi[b]).wait()

    def start_out(ci, b):
        row0 = wid * RW + ci * CH
        pltpu.async_copy(oc_b[b], vc_hbm.at[pl.ds(row0, CH)], semo[b])
        pltpu.async_copy(oo_b[b], vo_hbm.at[pl.ds(row0, CH)], semo[b])
        pltpu.async_copy(ol_b[b], vl_hbm.at[pl.ds(row0, CH)], semo[b])

    def wait_out(b):
        for buf, hbm in ((oc_b, vc_hbm), (oo_b, vo_hbm), (ol_b, vl_hbm)):
            pltpu.make_async_copy(buf[b], hbm.at[pl.ds(0, CH)],
                                  semo[b]).wait()

    _zero_ref(h_v, 3 * B1 * L)
    start_in(0, 0)

    @pl.loop(0, NCH // 2)
    def _(oc):
        for b in range(2):
            ci = oc * 2 + b
            wait_in(b)

            @pl.when(ci + 1 < NCH)
            def _():
                start_in(ci + 1, 1 - b)

            @pl.when(ci >= 2)
            def _():
                wait_out(b)

            @pl.loop(0, CH // L, unroll=4)
            def _(g):
                sl = pl.ds(g * L, L)
                lbl = lab_b[b][sl]
                a = a_b[b][sl]
                sg = s_b[b][sl]
                per_off = so_b[b][sl]
                per_lmk = sl_b[b][sl]

                is1 = lbl == 1
                per_cls = jnp.where(is1, a - sg, a)
                keep = lbl >= 0
                offm = is1 | (lbl == -1)
                lmkm = lbl == -2

                oc_b[b][sl] = jnp.where(keep, per_cls, neg1)
                oo_b[b][sl] = jnp.where(offm, per_off, neg1)
                ol_b[b][sl] = jnp.where(lmkm, per_lmk, neg1)

                for k, (per, msk) in enumerate(
                        ((per_cls, keep), (per_off, offm),
                         (per_lmk, lmkm))):
                    bits = plsc.bitcast(per, jnp.int32)
                    bb = lax.shift_right_logical(bits, c21)
                    idx = (k * B1 + bb) * L + lane
                    plsc.addupdate_scatter(h_v, [idx], ones, mask=msk)

            start_out(ci, b)

    for b in range(2):
        wait_out(b)

    for k in range(3):
        _lane_fold(h_v, k * B1 * L, fold_v, k * B1, B1, lane)
    _combine_per_sc(fold_v, shared, stage_v, acc_v, h1_hbm, HR1)


@functools.partial(
    pl.kernel,
    out_type=jax.ShapeDtypeStruct((NC * HR2,), jnp.float32),
    mesh=_MESH,
    compiler_params=_CPARAMS,
    scratch_types=(
        pltpu.VMEM((HR1,), jnp.float32),
        pltpu.VMEM((HR1,), jnp.float32),
        (pltpu.VMEM((CH,), jnp.float32),) * 2,
        (pltpu.VMEM((CH,), jnp.float32),) * 2,
        (pltpu.VMEM((CH,), jnp.float32),) * 2,
        pltpu.VMEM((6 * B2 * L,), jnp.float32),
        pltpu.VMEM((3 * L,), jnp.float32),
        pltpu.VMEM((HR2,), jnp.float32),
        pltpu.VMEM((HR2,), jnp.float32),
        pltpu.VMEM((HR2,), jnp.float32),
        pltpu.VMEM_SHARED((NS, HR2), jnp.float32),
        (pltpu.SemaphoreType.DMA,) * 2,
    ),
)
def _kernel_h2(vc_hbm, vo_hbm, vl_hbm, h1_hbm, h2_hbm,
               acc1_v, st1_v, bc_b, bo_b, bl_b, h2_v, sacc_v, fold_v,
               stage_v, acc_v, shared, semi):
    wid = _wid()
    lane = lax.iota(jnp.int32, L)
    ones = jnp.ones((L,), jnp.float32)
    zerov = jnp.zeros((L,), jnp.float32)
    c21 = jnp.full((L,), 21, jnp.int32)
    c12 = jnp.full((L,), 12, jnp.int32)

    def start_in(ci, b):
        row0 = wid * RW + ci * CH
        for buf, hbm in ((bc_b, vc_hbm), (bo_b, vo_hbm), (bl_b, vl_hbm)):
            pltpu.async_copy(hbm.at[pl.ds(row0, CH)], buf[b], semi[b])

    def wait_in(b):
        for buf, hbm in ((bc_b, vc_hbm), (bo_b, vo_hbm), (bl_b, vl_hbm)):
            pltpu.make_async_copy(hbm.at[pl.ds(0, CH)], buf[b],
                                  semi[b]).wait()

    _accum_rows(h1_hbm, st1_v, acc1_v, HR1, NC)

    b1s = []
    for k in range(3):
        count = _hist_count(acc1_v, k * B1, B1)
        nk = _n_keep(count)
        b1, _ = _scan_cnt(acc1_v, k * B1, B1, nk)
        b1s.append(jnp.full((L,), b1, jnp.int32))

    _zero_ref(h2_v, 6 * B2 * L)
    _zero_ref(sacc_v, 3 * L)
    start_in(0, 0)

    @pl.loop(0, NCH // 2)
    def _(oc):
        for b in range(2):
            ci = oc * 2 + b
            wait_in(b)

            @pl.when(ci + 1 < NCH)
            def _():
                start_in(ci + 1, 1 - b)

            @pl.loop(0, CH // L, unroll=2)
            def _(g):
                sl = pl.ds(g * L, L)
                for k, bufs in enumerate((bc_b, bo_b, bl_b)):
                    v = bufs[b][sl]
                    valid = v >= 0.0
                    bits = plsc.bitcast(v, jnp.int32)
                    lvl1 = lax.shift_right_logical(bits, c21)
                    m_gt = (lvl1 > b1s[k]) & valid
                    ks = pl.ds(k * L, L)
                    sacc_v[ks] = sacc_v[ks] + jnp.where(m_gt, v, zerov)
                    m_eq = lvl1 == b1s[k]
                    sub = jnp.bitwise_and(
                        lax.shift_right_logical(bits, c12), B2 - 1)
                    idx = (k * 2 * B2 + sub) * L + lane
                    plsc.addupdate_scatter(h2_v, [idx], ones, mask=m_eq)
                    plsc.addupdate_scatter(h2_v, [idx + B2 * L], v,
                                           mask=m_eq)

    for k in range(6):
        _lane_fold(h2_v, k * B2 * L, fold_v, k * B2, B2, lane)
    sv = jnp.where(lane == 0, jnp.sum(sacc_v[pl.ds(0, L)]), zerov)
    sv = sv + jnp.where(lane == 1, jnp.sum(sacc_v[pl.ds(L, L)]), zerov)
    sv = sv + jnp.where(lane == 2, jnp.sum(sacc_v[pl.ds(2 * L, L)]),
                        zerov)
    fold_v[pl.ds(6 * B2, L)] = sv
    _combine_per_sc(fold_v, shared, stage_v, acc_v, h2_hbm, HR2)


@functools.partial(
    pl.kernel,
    out_type=jax.ShapeDtypeStruct((8,), jnp.float32),
    mesh=_MESH,
    compiler_params=_CPARAMS,
    scratch_types=(
        pltpu.VMEM((HR1,), jnp.float32),
        pltpu.VMEM((HR2,), jnp.float32),
        pltpu.VMEM((HR1,), jnp.float32),
        pltpu.VMEM((HR2,), jnp.float32),
        pltpu.VMEM((16,), jnp.float32),
    ),
)
def _kernel_c(h1_hbm, h2_hbm, out_hbm, acc1_v, acc2_v, st1_v, st2_v, out_v):
    wid = _wid()

    @pl.when(wid == 0)
    def _():
        _accum_rows(h1_hbm, st1_v, acc1_v, HR1, NC)
        _accum_rows(h2_hbm, st2_v, acc2_v, HR2, NC)

        losses = []
        for k in range(3):
            count = _hist_count(acc1_v, k * B1, B1)
            nk = _n_keep(count)
            b1, c1 = _scan_cnt(acc1_v, k * B1, B1, nk)
            s1 = _scalar_at(acc2_v, 6 * B2 + k)
            r1 = nk - c1
            b2, s2, c2 = _scan_top(acc2_v, k * 2 * B2, (k * 2 + 1) * B2,
                                   B2, r1)
            r2 = r1 - c2
            cnt_b2 = _scalar_at(acc2_v, k * 2 * B2 + b2)
            sum_b2 = _scalar_at(acc2_v, (k * 2 + 1) * B2 + b2)
            part = jnp.where(r2 > 0.5, r2 * _sdiv(sum_b2, cnt_b2), 0.0)
            total = s1 + s2 + part
            mean = _sdiv(total, nk)
            if k == 0:
                losses.append(mean)
            else:
                losses.append(jnp.where(count < 0.5, 0.0, mean))

        loss_cls, loss_off, loss_lmk = losses
        loss_total = CLS_W * loss_cls + BBOX_W * loss_off + LMK_W * loss_lmk
        lane = lax.iota(jnp.int32, L)
        zeros = jnp.zeros((L,), jnp.float32)
        ov = jnp.where(lane == 0, loss_total, zeros)
        ov = ov + jnp.where(lane == 1, loss_cls, zeros)
        ov = ov + jnp.where(lane == 2, loss_off, zeros)
        ov = ov + jnp.where(lane == 3, loss_lmk, zeros)
        out_v[pl.ds(0, L)] = ov
        pltpu.sync_copy(out_v.at[pl.ds(0, 8)], out_hbm)


def kernel(pred, labels, offsets, landmarks):
    va, vs, vso, vsl = _tc_values(pred.T, offsets.T, landmarks.T)
    vc, vo, vl, h1 = _kernel_h1(labels, va, vs, vso, vsl)
    h2 = _kernel_h2(vc, vo, vl, h1)
    out = _kernel_c(h1, h2)
    return (out[0], out[1], out[2], out[3])


# speedup vs baseline: 15.4240x; 1.0010x over previous
import functools

import jax
import jax.numpy as jnp
import numpy as np
from jax import lax
from jax.experimental import pallas as pl
from jax.experimental.pallas import tpu as pltpu
from jax.experimental.pallas import tpu_sc as plsc

N = 1048576
NC = 2
NS = 16
NW = NC * NS
L = 16
RW = N // NW

RTC = 16384
CH = 4096
NCH = RW // CH

B1 = 1024
B2 = 512
HR1 = 3 * B1
HR2 = 3 * 2 * B2 + 16

CLS_W = 1.0
BBOX_W = 0.5
LMK_W = 0.5

_G_COEF = (1.8498544538905285e-04, 2.8751506391739456e-04,
           -5.4268610571399910e-03, 8.3107776364009530e-05,
           1.2498464620813230e-01, -4.9999884358222030e-01,
           6.9314715967354310e-01)

_MESH = plsc.VectorSubcoreMesh(core_axis_name="c", subcore_axis_name="s")
_CPARAMS = pltpu.CompilerParams(needs_layout_passes=False)


def _tc_body(pred_ref, off_ref, lmk_ref, va_ref, vs_ref, vo_ref, vl_ref):
    pt = pred_ref[...]
    ot = off_ref[...]
    lt = lmk_ref[...]
    do = pt[1:5, :] - ot
    dl = pt[5:15, :] - lt
    z = pt[0, :]
    sg = 1.0 / (1.0 + jnp.exp(-z))
    va_ref[...] = sg + _g_poly(sg)
    vs_ref[...] = sg
    vo_ref[...] = jnp.sum(do * do, axis=0) * 0.25
    vl_ref[...] = jnp.sum(dl * dl, axis=0) * 0.1


_tc_values = pl.pallas_call(
    _tc_body,
    grid=(N // RTC,),
    in_specs=[
        pl.BlockSpec((15, RTC), lambda i: (0, i)),
        pl.BlockSpec((4, RTC), lambda i: (0, i)),
        pl.BlockSpec((10, RTC), lambda i: (0, i)),
    ],
    out_specs=[pl.BlockSpec((RTC,), lambda i: (i,))] * 4,
    out_shape=[jax.ShapeDtypeStruct((N,), jnp.float32)] * 4,
)


def _wid():
    return lax.axis_index("s") * NC + lax.axis_index("c")


def _g_poly(s):
    acc = jnp.full(s.shape, _G_COEF[0], jnp.float32)
    for c in _G_COEF[1:]:
        acc = acc * s + c
    return acc


def _zero_ref(ref, nwords):
    z = jnp.zeros((L,), jnp.float32)

    @pl.loop(0, nwords // L)
    def _(i):
        ref[pl.ds(i * L, L)] = z


def _lane_fold(src, src_base, dst, dst_base, nbins, lane):
    @pl.loop(0, nbins // L)
    def _(i):
        bins = i * L + lane
        acc = jnp.zeros((L,), jnp.float32)
        for l in range(L):
            acc = acc + plsc.load_gather(src, [src_base + bins * L + l])
        dst[pl.ds(dst_base + i * L, L)] = acc


def _accum_rows(src_hbm, stage, acc, nwords, nrows):
    _zero_ref(acc, nwords)

    @pl.loop(0, nrows)
    def _(t):
        pltpu.sync_copy(src_hbm.at[pl.ds(t * nwords, nwords)], stage)

        @pl.loop(0, nwords // L)
        def _(i):
            sl = pl.ds(i * L, L)
            acc[sl] = acc[sl] + stage[sl]


def _combine_per_sc(fold_v, shared, stage, acc, out_hbm, nwords):
    sid = lax.axis_index("s")
    cid = lax.axis_index("c")
    pltpu.sync_copy(fold_v, shared.at[sid])
    plsc.subcore_barrier()

    @pl.when(sid == 0)
    def _():
        _zero_ref(acc, nwords)

        @pl.loop(0, NS)
        def _(t):
            pltpu.sync_copy(shared.at[t], stage)

            @pl.loop(0, nwords // L)
            def _(i):
                sl = pl.ds(i * L, L)
                acc[sl] = acc[sl] + stage[sl]

        pltpu.sync_copy(acc.at[pl.ds(0, nwords)],
                        out_hbm.at[pl.ds(cid * nwords, nwords)])


def _scan_top(ref, cnt_base, sum_base, nbins, target):
    nb = nbins // L

    def body(j, carry):
        found, b_star, s_above, c_above, ccnt, csum = carry
        vb = nb - 1 - j
        vc = ref[pl.ds(cnt_base + vb * L, L)]
        vs = ref[pl.ds(sum_base + vb * L, L)]
        rc = lax.rev(vc, (0,))
        rs = lax.rev(vs, (0,))
        cum = jnp.cumsum(rc) + ccnt
        m = cum >= target
        p = jnp.sum(jnp.where(m, 1.0, 0.0))
        has = (p > 0.5).astype(jnp.int32)
        b_here = vb * L + lax.convert_element_type(p, jnp.int32) - 1
        c_here = ccnt + jnp.sum(jnp.where(m, 0.0, rc))
        s_here = csum + jnp.sum(jnp.where(m, 0.0, rs))
        take = has * (1 - found)
        b_star = jnp.where(take > 0, b_here, b_star)
        s_above = jnp.where(take > 0, s_here, s_above)
        c_above = jnp.where(take > 0, c_here, c_above)
        found = jnp.maximum(found, has)
        ccnt = ccnt + jnp.sum(vc)
        csum = csum + jnp.sum(vs)
        return (found, b_star, s_above, c_above, ccnt, csum)

    init = (jnp.int32(0), jnp.int32(0), jnp.float32(0.0), jnp.float32(0.0),
            jnp.float32(0.0), jnp.float32(0.0))
    _, b_star, s_above, c_above, _, _ = lax.fori_loop(0, nb, body, init)
    return b_star, s_above, c_above


def _scan_cnt(ref, base, nbins, target):
    nb = nbins // L

    def body(j, carry):
        found, b_star, c_above, ccnt = carry
        vb = nb - 1 - j
        vc = ref[pl.ds(base + vb * L, L)]
        rc = lax.rev(vc, (0,))
        cum = jnp.cumsum(rc) + ccnt
        m = cum >= target
        p = jnp.sum(jnp.where(m, 1.0, 0.0))
        has = (p > 0.5).astype(jnp.int32)
        b_here = vb * L + lax.convert_element_type(p, jnp.int32) - 1
        c_here = ccnt + jnp.sum(jnp.where(m, 0.0, rc))
        take = has * (1 - found)
        b_star = jnp.where(take > 0, b_here, b_star)
        c_above = jnp.where(take > 0, c_here, c_above)
        found = jnp.maximum(found, has)
        ccnt = ccnt + jnp.sum(vc)
        return (found, b_star, c_above, ccnt)

    init = (jnp.int32(0), jnp.int32(0), jnp.float32(0.0), jnp.float32(0.0))
    _, b_star, c_above, _ = lax.fori_loop(0, nb, body, init)
    return b_star, c_above


def _hist_count(ref, cnt_base, nbins):
    acc = jnp.zeros((L,), jnp.float32)

    def body(i, acc):
        return acc + ref[pl.ds(cnt_base + i * L, L)]

    acc = lax.fori_loop(0, nbins // L, body, acc)
    return jnp.sum(acc)


def _n_keep(count_f):
    ci = lax.convert_element_type(count_f, jnp.int32)
    nk = (7 * ci) // 10
    return lax.convert_element_type(nk, jnp.float32)


def _sdiv(a, b):
    q = jnp.full((L,), a, jnp.float32) / jnp.full((L,), b, jnp.float32)
    lane = lax.iota(jnp.int32, L)
    return jnp.sum(jnp.where(lane == 0, q, jnp.zeros((L,), jnp.float32)))


def _scalar_at(ref, idx):
    v = plsc.load_gather(ref, [jnp.full((L,), idx, jnp.int32)])
    return jnp.sum(v) * (1.0 / L)


@functools.partial(
    pl.kernel,
    out_type=(
        jax.ShapeDtypeStruct((N,), jnp.float32),
        jax.ShapeDtypeStruct((N,), jnp.float32),
        jax.ShapeDtypeStruct((N,), jnp.float32),
        jax.ShapeDtypeStruct((NC * HR1,), jnp.float32),
    ),
    mesh=_MESH,
    compiler_params=_CPARAMS,
    scratch_types=(
        (pltpu.VMEM((CH,), jnp.int32),) * 2,
        (pltpu.VMEM((CH,), jnp.float32),) * 2,
        (pltpu.VMEM((CH,), jnp.float32),) * 2,
        (pltpu.VMEM((CH,), jnp.float32),) * 2,
        (pltpu.VMEM((CH,), jnp.float32),) * 2,
        (pltpu.VMEM((CH,), jnp.float32),) * 2,
        (pltpu.VMEM((CH,), jnp.float32),) * 2,
        (pltpu.VMEM((CH,), jnp.float32),) * 2,
        pltpu.VMEM((3 * B1 * L,), jnp.float32),
        pltpu.VMEM((HR1,), jnp.float32),
        pltpu.VMEM((HR1,), jnp.float32),
        pltpu.VMEM((HR1,), jnp.float32),
        pltpu.VMEM_SHARED((NS, HR1), jnp.float32),
        (pltpu.SemaphoreType.DMA,) * 2,
        (pltpu.SemaphoreType.DMA,) * 2,
    ),
)
def _kernel_h1(lab_hbm, va_hbm, vs_hbm, vso_hbm, vsl_hbm,
               vc_hbm, vo_hbm, vl_hbm, h1_hbm,
               lab_b, a_b, s_b, so_b, sl_b, oc_b, oo_b, ol_b,
               h_v, fold_v, stage_v, acc_v, shared, semi, semo):
    wid = _wid()
    lane = lax.iota(jnp.int32, L)
    ones = jnp.ones((L,), jnp.float32)
    neg1 = jnp.full((L,), -1.0, jnp.float32)
    c21 = jnp.full((L,), 21, jnp.int32)

    in_pairs = ((lab_hbm, lab_b), (va_hbm, a_b), (vs_hbm, s_b),
                (vso_hbm, so_b), (vsl_hbm, sl_b))

    def start_in(ci, b):
        row0 = wid * RW + ci * CH
        for hbm, buf in in_pairs:
            pltpu.async_copy(hbm.at[pl.ds(row0, CH)], buf[b], semi[b])

    def wait_in(b):
        for hbm, buf in in_pairs:
            pltpu.make_async_copy(hbm.at[pl.ds(0, CH)], buf[b],
                                  semi[b]).wait()

    def start_out(ci, b):
        row0 = wid * RW + ci * CH
        pltpu.async_copy(oc_b[b], vc_hbm.at[pl.ds(row0, CH)], semo[b])
        pltpu.async_copy(oo_b[b], vo_hbm.at[pl.ds(row0, CH)], semo[b])
        pltpu.async_copy(ol_b[b], vl_hbm.at[pl.ds(row0, CH)], semo[b])

    def wait_out(b):
        for buf, hbm in ((oc_b, vc_hbm), (oo_b, vo_hbm), (ol_b, vl_hbm)):
            pltpu.make_async_copy(buf[b], hbm.at[pl.ds(0, CH)],
                                  semo[b]).wait()

    _zero_ref(h_v, 3 * B1 * L)
    start_in(0, 0)

    @pl.loop(0, NCH // 2)
    def _(oc):
        for b in range(2):
            ci = oc * 2 + b
            wait_in(b)

            @pl.when(ci + 1 < NCH)
            def _():
                start_in(ci + 1, 1 - b)

            @pl.when(ci >= 2)
            def _():
                wait_out(b)

            @pl.loop(0, CH // L, unroll=8)
            def _(g):
                sl = pl.ds(g * L, L)
                lbl = lab_b[b][sl]
                a = a_b[b][sl]
                sg = s_b[b][sl]
                per_off = so_b[b][sl]
                per_lmk = sl_b[b][sl]

                is1 = lbl == 1
                per_cls = jnp.where(is1, a - sg, a)
                keep = lbl >= 0
                offm = is1 | (lbl == -1)
                lmkm = lbl == -2

                oc_b[b][sl] = jnp.where(keep, per_cls, neg1)
                oo_b[b][sl] = jnp.where(offm, per_off, neg1)
                ol_b[b][sl] = jnp.where(lmkm, per_lmk, neg1)

                for k, (per, msk) in enumerate(
                        ((per_cls, keep), (per_off, offm),
                         (per_lmk, lmkm))):
                    bits = plsc.bitcast(per, jnp.int32)
                    bb = lax.shift_right_logical(bits, c21)
                    idx = (k * B1 + bb) * L + lane
                    plsc.addupdate_scatter(h_v, [idx], ones, mask=msk)

            start_out(ci, b)

    for b in range(2):
        wait_out(b)

    for k in range(3):
        _lane_fold(h_v, k * B1 * L, fold_v, k * B1, B1, lane)
    _combine_per_sc(fold_v, shared, stage_v, acc_v, h1_hbm, HR1)


@functools.partial(
    pl.kernel,
    out_type=jax.ShapeDtypeStruct((NC * HR2,), jnp.float32),
    mesh=_MESH,
    compiler_params=_CPARAMS,
    scratch_types=(
        pltpu.VMEM((HR1,), jnp.float32),
        pltpu.VMEM((HR1,), jnp.float32),
        (pltpu.VMEM((CH,), jnp.float32),) * 2,
        (pltpu.VMEM((CH,), jnp.float32),) * 2,
        (pltpu.VMEM((CH,), jnp.float32),) * 2,
        pltpu.VMEM((6 * B2 * L,), jnp.float32),
        pltpu.VMEM((3 * L,), jnp.float32),
        pltpu.VMEM((HR2,), jnp.float32),
        pltpu.VMEM((HR2,), jnp.float32),
        pltpu.VMEM((HR2,), jnp.float32),
        pltpu.VMEM_SHARED((NS, HR2), jnp.float32),
        (pltpu.SemaphoreType.DMA,) * 2,
    ),
)
def _kernel_h2(vc_hbm, vo_hbm, vl_hbm, h1_hbm, h2_hbm,
               acc1_v, st1_v, bc_b, bo_b, bl_b, h2_v, sacc_v, fold_v,
               stage_v, acc_v, shared, semi):
    wid = _wid()
    lane = lax.iota(jnp.int32, L)
    ones = jnp.ones((L,), jnp.float32)
    zerov = jnp.zeros((L,), jnp.float32)
    c21 = jnp.full((L,), 21, jnp.int32)
    c12 = jnp.full((L,), 12, jnp.int32)

    def start_in(ci, b):
        row0 = wid * RW + ci * CH
        for buf, hbm in ((bc_b, vc_hbm), (bo_b, vo_hbm), (bl_b, vl_hbm)):
            pltpu.async_copy(hbm.at[pl.ds(row0, CH)], buf[b], semi[b])

    def wait_in(b):
        for buf, hbm in ((bc_b, vc_hbm), (bo_b, vo_hbm), (bl_b, vl_hbm)):
            pltpu.make_async_copy(hbm.at[pl.ds(0, CH)], buf[b],
                                  semi[b]).wait()

    _accum_rows(h1_hbm, st1_v, acc1_v, HR1, NC)

    b1s = []
    for k in range(3):
        count = _hist_count(acc1_v, k * B1, B1)
        nk = _n_keep(count)
        b1, _ = _scan_cnt(acc1_v, k * B1, B1, nk)
        b1s.append(jnp.full((L,), b1, jnp.int32))

    _zero_ref(h2_v, 6 * B2 * L)
    _zero_ref(sacc_v, 3 * L)
    start_in(0, 0)

    @pl.loop(0, NCH // 2)
    def _(oc):
        for b in range(2):
            ci = oc * 2 + b
            wait_in(b)

            @pl.when(ci + 1 < NCH)
            def _():
                start_in(ci + 1, 1 - b)

            @pl.loop(0, CH // L, unroll=8)
            def _(g):
                sl = pl.ds(g * L, L)
                for k, bufs in enumerate((bc_b, bo_b, bl_b)):
                    v = bufs[b][sl]
                    valid = v >= 0.0
                    bits = plsc.bitcast(v, jnp.int32)
                    lvl1 = lax.shift_right_logical(bits, c21)
                    m_gt = (lvl1 > b1s[k]) & valid
                    ks = pl.ds(k * L, L)
                    sacc_v[ks] = sacc_v[ks] + jnp.where(m_gt, v, zerov)
                    m_eq = lvl1 == b1s[k]
                    sub = jnp.bitwise_and(
                        lax.shift_right_logical(bits, c12), B2 - 1)
                    idx = (k * 2 * B2 + sub) * L + lane
                    plsc.addupdate_scatter(h2_v, [idx], ones, mask=m_eq)
                    plsc.addupdate_scatter(h2_v, [idx + B2 * L], v,
                                           mask=m_eq)

    for k in range(6):
        _lane_fold(h2_v, k * B2 * L, fold_v, k * B2, B2, lane)
    sv = jnp.where(lane == 0, jnp.sum(sacc_v[pl.ds(0, L)]), zerov)
    sv = sv + jnp.where(lane == 1, jnp.sum(sacc_v[pl.ds(L, L)]), zerov)
    sv = sv + jnp.where(lane == 2, jnp.sum(sacc_v[pl.ds(2 * L, L)]),
                        zerov)
    fold_v[pl.ds(6 * B2, L)] = sv
    _combine_per_sc(fold_v, shared, stage_v, acc_v, h2_hbm, HR2)


@functools.partial(
    pl.kernel,
    out_type=jax.ShapeDtypeStruct((8,), jnp.float32),
    mesh=_MESH,
    compiler_params=_CPARAMS,
    scratch_types=(
        pltpu.VMEM((HR1,), jnp.float32),
        pltpu.VMEM((HR2,), jnp.float32),
        pltpu.VMEM((HR1,), jnp.float32),
        pltpu.VMEM((HR2,), jnp.float32),
        pltpu.VMEM((16,), jnp.float32),
    ),
)
def _kernel_c(h1_hbm, h2_hbm, out_hbm, acc1_v, acc2_v, st1_v, st2_v, out_v):
    wid = _wid()

    @pl.when(wid == 0)
    def _():
        _accum_rows(h1_hbm, st1_v, acc1_v, HR1, NC)
        _accum_rows(h2_hbm, st2_v, acc2_v, HR2, NC)

        losses = []
        for k in range(3):
            count = _hist_count(acc1_v, k * B1, B1)
            nk = _n_keep(count)
            b1, c1 = _scan_cnt(acc1_v, k * B1, B1, nk)
            s1 = _scalar_at(acc2_v, 6 * B2 + k)
            r1 = nk - c1
            b2, s2, c2 = _scan_top(acc2_v, k * 2 * B2, (k * 2 + 1) * B2,
                                   B2, r1)
            r2 = r1 - c2
            cnt_b2 = _scalar_at(acc2_v, k * 2 * B2 + b2)
            sum_b2 = _scalar_at(acc2_v, (k * 2 + 1) * B2 + b2)
            part = jnp.where(r2 > 0.5, r2 * _sdiv(sum_b2, cnt_b2), 0.0)
            total = s1 + s2 + part
            mean = _sdiv(total, nk)
            if k == 0:
                losses.append(mean)
            else:
                losses.append(jnp.where(count < 0.5, 0.0, mean))

        loss_cls, loss_off, loss_lmk = losses
        loss_total = CLS_W * loss_cls + BBOX_W * loss_off + LMK_W * loss_lmk
        lane = lax.iota(jnp.int32, L)
        zeros = jnp.zeros((L,), jnp.float32)
        ov = jnp.where(lane == 0, loss_total, zeros)
        ov = ov + jnp.where(lane == 1, loss_cls, zeros)
        ov = ov + jnp.where(lane == 2, loss_off, zeros)
        ov = ov + jnp.where(lane == 3, loss_lmk, zeros)
        out_v[pl.ds(0, L)] = ov
        pltpu.sync_copy(out_v.at[pl.ds(0, 8)], out_hbm)


def kernel(pred, labels, offsets, landmarks):
    va, vs, vso, vsl = _tc_values(pred.T, offsets.T, landmarks.T)
    vc, vo, vl, h1 = _kernel_h1(labels, va, vs, vso, vsl)
    h2 = _kernel_h2(vc, vo, vl, h1)
    out = _kernel_c(h1, h2)
    return (out[0], out[1], out[2], out[3])
